# baseline (speedup 1.0000x reference)
import jax
import jax.numpy as jnp
from jax import lax
from jax.experimental import pallas as pl
from jax.experimental.pallas import tpu as pltpu
from jax.experimental.pallas import tpu_sc as plsc

_NC = 2
_NS = 16
_NW = _NC * _NS

_SB = 64
_NSUB = 8
_GB = _SB * _NSUB
_WB = 624

_PARAMS = pltpu.CompilerParams(use_tc_tiling_on_sc=False)


def _mesh():
    return plsc.VectorSubcoreMesh(core_axis_name="c", subcore_axis_name="s")


def _slabs(N):
    out = [(0, _WB)]
    tail0 = _NS * _WB
    if tail0 < N:
        out.append((tail0, N - tail0))
    return out


def _make_prop(N, E, H):
    assert E % _GB == 0
    ncht = E // _GB
    nch = (ncht + _NW - 1) // _NW
    assert _NS * _WB <= N <= _NS * _WB + _WB and (N - _NS * _WB) % 8 == 0

    nbuf = 2
    nu = (nch + nbuf - 1) // nbuf

    def body(y_hbm, row_hbm, col_hbm, out_hbm, gidx, sidx, rows_v, acc_sh,
             sem, sems):
        ci = lax.axis_index("c")
        si = lax.axis_index("s")
        wid = si * _NC + ci

        def zrow(i, _):
            for j in range(H // 16):
                rows_v[0][i, pl.ds(j * 16, 16)] = jnp.zeros((16,), jnp.float32)
            return _
        lax.fori_loop(0, _GB, zrow, None)
        for base, n in _slabs(N):
            if base > 0:
                @pl.when(si == 0)
                def _():
                    pltpu.sync_copy(rows_v[0].at[pl.ds(0, n)],
                                    acc_sh.at[pl.ds(base, n)])
            else:
                done = 0
                while done < _WB:
                    m = min(_GB, _WB - done)
                    off = pl.multiple_of(si * _WB + done, 8)
                    pltpu.sync_copy(rows_v[0].at[pl.ds(0, m)],
                                    acc_sh.at[pl.ds(off, m)])
                    done += m
        plsc.subcore_barrier()

        def drain(b, pred):
            @pl.when(pred)
            def _():
                for j in range(_NSUB):
                    pltpu.make_async_copy(
                        rows_v[b].at[pl.ds(j * _SB, _SB)],
                        acc_sh.at[sidx[b][j]], sems[b]).wait()

        def step(u, _):
            for b in range(nbuf):
                c = (u * nbuf + b) * _NW + wid
                cprev = c - nbuf * _NW
                drain(b, (u > 0) & (cprev < ncht))

                @pl.when(c < ncht)
                def _():
                    eoff = pl.multiple_of(c * _GB, 8)
                    hs = [pltpu.async_copy(
                              row_hbm.at[pl.ds(eoff + j * _SB, _SB)],
                              gidx[b][j], sem)
                          for j in range(_NSUB)]
                    hs += [pltpu.async_copy(
                               col_hbm.at[pl.ds(eoff + j * _SB, _SB)],
                               sidx[b][j], sem)
                           for j in range(_NSUB)]
                    for h in hs:
                        h.wait()
                    gs = [pltpu.async_copy(y_hbm.at[gidx[b][j]],
                                           rows_v[b].at[pl.ds(j * _SB, _SB)],
                                           sem)
                          for j in range(_NSUB)]
                    for g in gs:
                        g.wait()
                    for j in range(_NSUB):
                        pltpu.async_copy(rows_v[b].at[pl.ds(j * _SB, _SB)],
                                         acc_sh.at[sidx[b][j]], sems[b],
                                         add=True)
            return _
        lax.fori_loop(0, nu, step, None)
        for b in range(nbuf):
            clast = ((nu - 1) * nbuf + b) * _NW + wid
            drain(b, clast < ncht)
        plsc.subcore_barrier()

        for base, n in _slabs(N):
            if base > 0:
                @pl.when(si == 0)
                def _():
                    pltpu.sync_copy(acc_sh.at[pl.ds(base, n)],
                                    rows_v[0].at[pl.ds(0, n)])
                    pltpu.sync_copy(rows_v[0].at[pl.ds(0, n)],
                                    out_hbm.at[pl.ds(ci * N + base, n)])
            else:
                done = 0
                while done < _WB:
                    m = min(_GB, _WB - done)
                    off = pl.multiple_of(si * _WB + done, 8)
                    pltpu.sync_copy(acc_sh.at[pl.ds(off, m)],
                                    rows_v[0].at[pl.ds(0, m)])
                    pltpu.sync_copy(rows_v[0].at[pl.ds(0, m)],
                                    out_hbm.at[pl.ds(ci * N + off, m)])
                    done += m

    return pl.kernel(
        body,
        out_type=jax.ShapeDtypeStruct((_NC * N, H), jnp.float32),
        mesh=_mesh(),
        compiler_params=_PARAMS,
        scratch_types=[
            [[pltpu.VMEM((_SB,), jnp.int32) for _ in range(_NSUB)]
             for _ in range(nbuf)],
            [[pltpu.VMEM((_SB,), jnp.int32) for _ in range(_NSUB)]
             for _ in range(nbuf)],
            [pltpu.VMEM((_GB, H), jnp.float32) for _ in range(nbuf)],
            pltpu.VMEM_SHARED((N, H), jnp.float32),
            pltpu.SemaphoreType.DMA,
            [pltpu.SemaphoreType.DMA for _ in range(nbuf)],
        ],
    )


def _make_deg(N, E):
    H = 16
    assert E % _GB == 0
    ncht = E // _GB
    nch = (ncht + _NW - 1) // _NW

    def body(row_hbm, out_hbm, sidx, buf_v, acc_sh, sem):
        ci = lax.axis_index("c")
        si = lax.axis_index("s")
        wid = si * _NC + ci

        def zrow(i, _):
            buf_v[i, pl.ds(0, 16)] = jnp.zeros((16,), jnp.float32)
            return _
        lax.fori_loop(0, _WB, zrow, None)
        for base, n in _slabs(N):
            if base > 0:
                @pl.when(si == 0)
                def _():
                    pltpu.sync_copy(buf_v.at[pl.ds(0, n)],
                                    acc_sh.at[pl.ds(base, n)])
            else:
                off = pl.multiple_of(si * _WB, 8)
                pltpu.sync_copy(buf_v.at[pl.ds(0, _WB)],
                                acc_sh.at[pl.ds(off, _WB)])

        def onerow(i, _):
            buf_v[i, pl.ds(0, 16)] = jnp.ones((16,), jnp.float32)
            return _
        lax.fori_loop(0, _SB, onerow, None)
        plsc.subcore_barrier()

        def chunk(t, _):
            c = t * _NW + wid

            @pl.when(c < ncht)
            def _():
                eoff = pl.multiple_of(c * _GB, 8)
                for j in range(_NSUB):
                    pltpu.sync_copy(row_hbm.at[pl.ds(eoff + j * _SB, _SB)],
                                    sidx[j])
                for j in range(_NSUB):
                    pltpu.sync_copy(buf_v.at[pl.ds(0, _SB)],
                                    acc_sh.at[sidx[j]], add=True)
            return _
        lax.fori_loop(0, nch, chunk, None)
        plsc.subcore_barrier()

        for base, n in _slabs(N):
            if base > 0:
                @pl.when(si == 0)
                def _():
                    pltpu.sync_copy(acc_sh.at[pl.ds(base, n)],
                                    buf_v.at[pl.ds(0, n)])
                    pltpu.sync_copy(buf_v.at[pl.ds(0, n)],
                                    out_hbm.at[pl.ds(ci * N + base, n)])
            else:
                off = pl.multiple_of(si * _WB, 8)
                pltpu.sync_copy(acc_sh.at[pl.ds(off, _WB)],
                                buf_v.at[pl.ds(0, _WB)])
                pltpu.sync_copy(buf_v.at[pl.ds(0, _WB)],
                                out_hbm.at[pl.ds(ci * N + off, _WB)])

    return pl.kernel(
        body,
        out_type=jax.ShapeDtypeStruct((_NC * N, H), jnp.float32),
        mesh=_mesh(),
        compiler_params=_PARAMS,
        scratch_types=[
            [pltpu.VMEM((_SB,), jnp.int32) for _ in range(_NSUB)],
            pltpu.VMEM((_WB, H), jnp.float32),
            pltpu.VMEM_SHARED((N, H), jnp.float32),
            pltpu.SemaphoreType.DMA,
        ],
    )


_BLK = 1000


def _tc_dense1(N, D, H, K1):
    def body(x_ref, w_ref, o_ref):
        for k in range(K1):
            o_ref[k] = jnp.dot(x_ref[...], w_ref[k],
                               preferred_element_type=jnp.float32)
    return pl.pallas_call(
        body,
        grid=(N // _BLK,),
        in_specs=[pl.BlockSpec((_BLK, D), lambda i: (i, 0)),
                  pl.BlockSpec((K1, D, H), lambda i: (0, 0, 0))],
        out_specs=pl.BlockSpec((K1, _BLK, H), lambda i: (0, i, 0)),
        out_shape=jax.ShapeDtypeStruct((K1, N, H), jnp.float32),
    )


def _tc_dis(N, nb):
    def body(pt_ref, pb_ref, o_ref):
        deg = pt_ref[...][:, :1] + pb_ref[...][:, :1]
        o_ref[...] = jnp.where(deg > 0, lax.rsqrt(deg), 0.0)
    return pl.pallas_call(
        body,
        grid=(N // _BLK,),
        in_specs=[pl.BlockSpec((_BLK, 16), lambda i: (i, 0)),
                  pl.BlockSpec((_BLK, 16), lambda i: (i + nb, 0))],
        out_specs=pl.BlockSpec((_BLK, 1), lambda i: (i, 0)),
        out_shape=jax.ShapeDtypeStruct((N, 1), jnp.float32),
    )


def _tc_scale(N, H, kcol):
    def body(z_ref, d_ref, o_ref):
        o_ref[...] = d_ref[...] * z_ref[0]
    return pl.pallas_call(
        body,
        grid=(N // _BLK,),
        in_specs=[pl.BlockSpec((1, _BLK, H), lambda i: (kcol, i, 0)),
                  pl.BlockSpec((_BLK, 1), lambda i: (i, 0))],
        out_specs=pl.BlockSpec((_BLK, H), lambda i: (i, 0)),
        out_shape=jax.ShapeDtypeStruct((N, H), jnp.float32),
    )


def _tc_combine(N, H, nb, kcol):
    def body(z_ref, pt_ref, pb_ref, b2_ref, d_ref, obk_ref, oy_ref):
        u = d_ref[...] * (pt_ref[...] + pb_ref[...])
        bk = z_ref[0] - 2.0 * u - b2_ref[...]
        obk_ref[...] = bk
        oy_ref[...] = d_ref[...] * bk
    return pl.pallas_call(
        body,
        grid=(N // _BLK,),
        in_specs=[pl.BlockSpec((1, _BLK, H), lambda i: (kcol, i, 0)),
                  pl.BlockSpec((_BLK, H), lambda i: (i, 0)),
                  pl.BlockSpec((_BLK, H), lambda i: (i + nb, 0)),
                  pl.BlockSpec((_BLK, H), lambda i: (i, 0)),
                  pl.BlockSpec((_BLK, 1), lambda i: (i, 0))],
        out_specs=[pl.BlockSpec((_BLK, H), lambda i: (i, 0)),
                   pl.BlockSpec((_BLK, H), lambda i: (i, 0))],
        out_shape=[jax.ShapeDtypeStruct((N, H), jnp.float32),
                   jax.ShapeDtypeStruct((N, H), jnp.float32)],
    )


def _tc_final(N, H, C, nb):
    def body(z_ref, pt_ref, pb_ref, b2k_ref, d_ref, b1_ref, w2_ref, bb_ref,
             o_ref):
        hpre = (z_ref[0] - d_ref[...] * (pt_ref[...] + pb_ref[...])
                - b2k_ref[...] + b1_ref[...])
        h = jnp.maximum(hpre, 0.0)
        logits = jnp.dot(h, w2_ref[...],
                         preferred_element_type=jnp.float32) + bb_ref[...]
        m = jnp.max(logits, axis=1, keepdims=True)
        e = jnp.exp(logits - m)
        o_ref[...] = e / jnp.sum(e, axis=1, keepdims=True)
    return pl.pallas_call(
        body,
        grid=(N // _BLK,),
        in_specs=[pl.BlockSpec((1, _BLK, H), lambda i: (0, i, 0)),
                  pl.BlockSpec((_BLK, H), lambda i: (i, 0)),
                  pl.BlockSpec((_BLK, H), lambda i: (i + nb, 0)),
                  pl.BlockSpec((_BLK, H), lambda i: (i, 0)),
                  pl.BlockSpec((_BLK, 1), lambda i: (i, 0)),
                  pl.BlockSpec((1, H), lambda i: (0, 0)),
                  pl.BlockSpec((H, C), lambda i: (0, 0)),
                  pl.BlockSpec((1, C), lambda i: (0, 0))],
        out_specs=pl.BlockSpec((_BLK, C), lambda i: (i, 0)),
        out_shape=jax.ShapeDtypeStruct((N, C), jnp.float32),
    )


def kernel(x, edge_index, W1, b1, W2, b2):
    N, D = x.shape
    E = edge_index.shape[1]
    K1, _, H = W1.shape
    C = W2.shape[2]
    KH = K1 * H
    nb = N // _BLK
    row, col = edge_index[0], edge_index[1]

    z3d = _tc_dense1(N, D, H, K1)(x, W1)

    partd = _make_deg(N, E)(row)
    dis = _tc_dis(N, nb)(partd, partd)

    prop_call = _make_prop(N, E, H)

    bk2 = jnp.zeros((N, H), jnp.float32)
    y = _tc_scale(N, H, K1 - 1)(z3d, dis)
    prev = z3d[K1 - 1]
    for k in range(K1 - 2, 0, -1):
        p = prop_call(y, row, col)
        bk, y = _tc_combine(N, H, nb, k)(z3d, p, p, bk2, dis)
        bk2 = prev
        prev = bk
    p = prop_call(y, row, col)
    return _tc_final(N, H, C, nb)(
        z3d, p, p, bk2, dis, b1.reshape(1, H), W2[0], b2.reshape(1, C))


# speedup vs baseline: 1.1031x; 1.1031x over previous
import jax
import jax.numpy as jnp
from jax import lax
from jax.experimental import pallas as pl
from jax.experimental.pallas import tpu as pltpu
from jax.experimental.pallas import tpu_sc as plsc

_NC = 2
_NS = 16
_NW = _NC * _NS

_SB = 64
_NSUB = 8
_GB = _SB * _NSUB
_WB = 624

_PARAMS = pltpu.CompilerParams(use_tc_tiling_on_sc=False)


def _mesh():
    return plsc.VectorSubcoreMesh(core_axis_name="c", subcore_axis_name="s")


def _slabs(N):
    out = [(0, _WB)]
    tail0 = _NS * _WB
    if tail0 < N:
        out.append((tail0, N - tail0))
    return out


def _make_prop(N, E, H):
    assert E % _GB == 0
    ncht = E // _GB
    nch = (ncht + _NW - 1) // _NW
    assert _NS * _WB <= N <= _NS * _WB + _WB and (N - _NS * _WB) % 8 == 0

    nbuf = 2
    nu = (nch + nbuf - 1) // nbuf

    def body(y_hbm, row_hbm, col_hbm, out_hbm, gidx, sidx, rows_v, acc_sh,
             sem, sems):
        ci = lax.axis_index("c")
        si = lax.axis_index("s")
        wid = si * _NC + ci

        def zrow(i, _):
            for j in range(H // 16):
                rows_v[0][i, pl.ds(j * 16, 16)] = jnp.zeros((16,), jnp.float32)
            return _
        lax.fori_loop(0, _GB, zrow, None)
        for base, n in _slabs(N):
            if base > 0:
                @pl.when(si == 0)
                def _():
                    pltpu.sync_copy(rows_v[0].at[pl.ds(0, n)],
                                    acc_sh.at[pl.ds(base, n)])
            else:
                done = 0
                while done < _WB:
                    m = min(_GB, _WB - done)
                    off = pl.multiple_of(si * _WB + done, 8)
                    pltpu.sync_copy(rows_v[0].at[pl.ds(0, m)],
                                    acc_sh.at[pl.ds(off, m)])
                    done += m
        plsc.subcore_barrier()

        def drain(b, pred):
            @pl.when(pred)
            def _():
                for j in range(_NSUB):
                    pltpu.make_async_copy(
                        rows_v[b].at[pl.ds(j * _SB, _SB)],
                        acc_sh.at[sidx[b][j]], sems[b]).wait()

        def step(u, _):
            for b in range(nbuf):
                c = (u * nbuf + b) * _NW + wid
                cprev = c - nbuf * _NW
                drain(b, (u > 0) & (cprev < ncht))

                @pl.when(c < ncht)
                def _():
                    eoff = pl.multiple_of(c * _GB, 8)
                    hs = [pltpu.async_copy(
                              row_hbm.at[pl.ds(eoff + j * _SB, _SB)],
                              gidx[b][j], sem)
                          for j in range(_NSUB)]
                    hs += [pltpu.async_copy(
                               col_hbm.at[pl.ds(eoff + j * _SB, _SB)],
                               sidx[b][j], sem)
                           for j in range(_NSUB)]
                    for h in hs:
                        h.wait()
                    gs = [pltpu.async_copy(y_hbm.at[gidx[b][j]],
                                           rows_v[b].at[pl.ds(j * _SB, _SB)],
                                           sem)
                          for j in range(_NSUB)]
                    for j in range(_NSUB):
                        gs[j].wait()
                        pltpu.async_copy(rows_v[b].at[pl.ds(j * _SB, _SB)],
                                         acc_sh.at[sidx[b][j]], sems[b],
                                         add=True)
            return _
        lax.fori_loop(0, nu, step, None)
        for b in range(nbuf):
            clast = ((nu - 1) * nbuf + b) * _NW + wid
            drain(b, clast < ncht)
        plsc.subcore_barrier()

        for base, n in _slabs(N):
            if base > 0:
                @pl.when(si == 0)
                def _():
                    pltpu.sync_copy(acc_sh.at[pl.ds(base, n)],
                                    rows_v[0].at[pl.ds(0, n)])
                    pltpu.sync_copy(rows_v[0].at[pl.ds(0, n)],
                                    out_hbm.at[pl.ds(ci * N + base, n)])
            else:
                done = 0
                while done < _WB:
                    m = min(_GB, _WB - done)
                    off = pl.multiple_of(si * _WB + done, 8)
                    pltpu.sync_copy(acc_sh.at[pl.ds(off, m)],
                                    rows_v[0].at[pl.ds(0, m)])
                    pltpu.sync_copy(rows_v[0].at[pl.ds(0, m)],
                                    out_hbm.at[pl.ds(ci * N + off, m)])
                    done += m

    return pl.kernel(
        body,
        out_type=jax.ShapeDtypeStruct((_NC * N, H), jnp.float32),
        mesh=_mesh(),
        compiler_params=_PARAMS,
        scratch_types=[
            [[pltpu.VMEM((_SB,), jnp.int32) for _ in range(_NSUB)]
             for _ in range(nbuf)],
            [[pltpu.VMEM((_SB,), jnp.int32) for _ in range(_NSUB)]
             for _ in range(nbuf)],
            [pltpu.VMEM((_GB, H), jnp.float32) for _ in range(nbuf)],
            pltpu.VMEM_SHARED((N, H), jnp.float32),
            pltpu.SemaphoreType.DMA,
            [pltpu.SemaphoreType.DMA for _ in range(nbuf)],
        ],
    )


def _make_deg(N, E):
    H = 16
    assert E % _GB == 0
    ncht = E // _GB
    nch = (ncht + _NW - 1) // _NW

    nbuf = 2
    nu = (nch + nbuf - 1) // nbuf

    def body(row_hbm, out_hbm, sidx, buf_v, acc_sh, sem, sems):
        ci = lax.axis_index("c")
        si = lax.axis_index("s")
        wid = si * _NC + ci

        def zrow(i, _):
            buf_v[i, pl.ds(0, 16)] = jnp.zeros((16,), jnp.float32)
            return _
        lax.fori_loop(0, _WB, zrow, None)
        for base, n in _slabs(N):
            if base > 0:
                @pl.when(si == 0)
                def _():
                    pltpu.sync_copy(buf_v.at[pl.ds(0, n)],
                                    acc_sh.at[pl.ds(base, n)])
            else:
                off = pl.multiple_of(si * _WB, 8)
                pltpu.sync_copy(buf_v.at[pl.ds(0, _WB)],
                                acc_sh.at[pl.ds(off, _WB)])

        def onerow(i, _):
            buf_v[i, pl.ds(0, 16)] = jnp.ones((16,), jnp.float32)
            return _
        lax.fori_loop(0, _SB, onerow, None)
        plsc.subcore_barrier()

        def drain(b, pred):
            @pl.when(pred)
            def _():
                for j in range(_NSUB):
                    pltpu.make_async_copy(buf_v.at[pl.ds(0, _SB)],
                                          acc_sh.at[sidx[b][j]],
                                          sems[b]).wait()

        def step(u, _):
            for b in range(nbuf):
                c = (u * nbuf + b) * _NW + wid
                cprev = c - nbuf * _NW
                drain(b, (u > 0) & (cprev < ncht))

                @pl.when(c < ncht)
                def _():
                    eoff = pl.multiple_of(c * _GB, 8)
                    hs = [pltpu.async_copy(
                              row_hbm.at[pl.ds(eoff + j * _SB, _SB)],
                              sidx[b][j], sem)
                          for j in range(_NSUB)]
                    for j in range(_NSUB):
                        hs[j].wait()
                        pltpu.async_copy(buf_v.at[pl.ds(0, _SB)],
                                         acc_sh.at[sidx[b][j]], sems[b],
                                         add=True)
            return _
        lax.fori_loop(0, nu, step, None)
        for b in range(nbuf):
            clast = ((nu - 1) * nbuf + b) * _NW + wid
            drain(b, clast < ncht)
        plsc.subcore_barrier()

        for base, n in _slabs(N):
            if base > 0:
                @pl.when(si == 0)
                def _():
                    pltpu.sync_copy(acc_sh.at[pl.ds(base, n)],
                                    buf_v.at[pl.ds(0, n)])
                    pltpu.sync_copy(buf_v.at[pl.ds(0, n)],
                                    out_hbm.at[pl.ds(ci * N + base, n)])
            else:
                off = pl.multiple_of(si * _WB, 8)
                pltpu.sync_copy(acc_sh.at[pl.ds(off, _WB)],
                                buf_v.at[pl.ds(0, _WB)])
                pltpu.sync_copy(buf_v.at[pl.ds(0, _WB)],
                                out_hbm.at[pl.ds(ci * N + off, _WB)])

    return pl.kernel(
        body,
        out_type=jax.ShapeDtypeStruct((_NC * N, H), jnp.float32),
        mesh=_mesh(),
        compiler_params=_PARAMS,
        scratch_types=[
            [[pltpu.VMEM((_SB,), jnp.int32) for _ in range(_NSUB)]
             for _ in range(nbuf)],
            pltpu.VMEM((_WB, H), jnp.float32),
            pltpu.VMEM_SHARED((N, H), jnp.float32),
            pltpu.SemaphoreType.DMA,
            [pltpu.SemaphoreType.DMA for _ in range(nbuf)],
        ],
    )


_BLK = 1000


def _tc_dense1(N, D, H, K1):
    def body(x_ref, w_ref, o_ref):
        for k in range(K1):
            o_ref[k] = jnp.dot(x_ref[...], w_ref[k],
                               preferred_element_type=jnp.float32)
    return pl.pallas_call(
        body,
        grid=(N // _BLK,),
        in_specs=[pl.BlockSpec((_BLK, D), lambda i: (i, 0)),
                  pl.BlockSpec((K1, D, H), lambda i: (0, 0, 0))],
        out_specs=pl.BlockSpec((K1, _BLK, H), lambda i: (0, i, 0)),
        out_shape=jax.ShapeDtypeStruct((K1, N, H), jnp.float32),
    )


def _tc_dis(N, nb):
    def body(pt_ref, pb_ref, o_ref):
        deg = pt_ref[...][:, :1] + pb_ref[...][:, :1]
        o_ref[...] = jnp.where(deg > 0, lax.rsqrt(deg), 0.0)
    return pl.pallas_call(
        body,
        grid=(N // _BLK,),
        in_specs=[pl.BlockSpec((_BLK, 16), lambda i: (i, 0)),
                  pl.BlockSpec((_BLK, 16), lambda i: (i + nb, 0))],
        out_specs=pl.BlockSpec((_BLK, 1), lambda i: (i, 0)),
        out_shape=jax.ShapeDtypeStruct((N, 1), jnp.float32),
    )


def _tc_scale(N, H, kcol):
    def body(z_ref, d_ref, o_ref):
        o_ref[...] = d_ref[...] * z_ref[0]
    return pl.pallas_call(
        body,
        grid=(N // _BLK,),
        in_specs=[pl.BlockSpec((1, _BLK, H), lambda i: (kcol, i, 0)),
                  pl.BlockSpec((_BLK, 1), lambda i: (i, 0))],
        out_specs=pl.BlockSpec((_BLK, H), lambda i: (i, 0)),
        out_shape=jax.ShapeDtypeStruct((N, H), jnp.float32),
    )


def _tc_combine(N, H, nb, kcol):
    def body(z_ref, pt_ref, pb_ref, b2_ref, d_ref, obk_ref, oy_ref):
        u = d_ref[...] * (pt_ref[...] + pb_ref[...])
        bk = z_ref[0] - 2.0 * u - b2_ref[...]
        obk_ref[...] = bk
        oy_ref[...] = d_ref[...] * bk
    return pl.pallas_call(
        body,
        grid=(N // _BLK,),
        in_specs=[pl.BlockSpec((1, _BLK, H), lambda i: (kcol, i, 0)),
                  pl.BlockSpec((_BLK, H), lambda i: (i, 0)),
                  pl.BlockSpec((_BLK, H), lambda i: (i + nb, 0)),
                  pl.BlockSpec((_BLK, H), lambda i: (i, 0)),
                  pl.BlockSpec((_BLK, 1), lambda i: (i, 0))],
        out_specs=[pl.BlockSpec((_BLK, H), lambda i: (i, 0)),
                   pl.BlockSpec((_BLK, H), lambda i: (i, 0))],
        out_shape=[jax.ShapeDtypeStruct((N, H), jnp.float32),
                   jax.ShapeDtypeStruct((N, H), jnp.float32)],
    )


def _tc_final(N, H, C, nb):
    def body(z_ref, pt_ref, pb_ref, b2k_ref, d_ref, b1_ref, w2_ref, bb_ref,
             o_ref):
        hpre = (z_ref[0] - d_ref[...] * (pt_ref[...] + pb_ref[...])
                - b2k_ref[...] + b1_ref[...])
        h = jnp.maximum(hpre, 0.0)
        logits = jnp.dot(h, w2_ref[...],
                         preferred_element_type=jnp.float32) + bb_ref[...]
        m = jnp.max(logits, axis=1, keepdims=True)
        e = jnp.exp(logits - m)
        o_ref[...] = e / jnp.sum(e, axis=1, keepdims=True)
    return pl.pallas_call(
        body,
        grid=(N // _BLK,),
        in_specs=[pl.BlockSpec((1, _BLK, H), lambda i: (0, i, 0)),
                  pl.BlockSpec((_BLK, H), lambda i: (i, 0)),
                  pl.BlockSpec((_BLK, H), lambda i: (i + nb, 0)),
                  pl.BlockSpec((_BLK, H), lambda i: (i, 0)),
                  pl.BlockSpec((_BLK, 1), lambda i: (i, 0)),
                  pl.BlockSpec((1, H), lambda i: (0, 0)),
                  pl.BlockSpec((H, C), lambda i: (0, 0)),
                  pl.BlockSpec((1, C), lambda i: (0, 0))],
        out_specs=pl.BlockSpec((_BLK, C), lambda i: (i, 0)),
        out_shape=jax.ShapeDtypeStruct((N, C), jnp.float32),
    )


def kernel(x, edge_index, W1, b1, W2, b2):
    N, D = x.shape
    E = edge_index.shape[1]
    K1, _, H = W1.shape
    C = W2.shape[2]
    KH = K1 * H
    nb = N // _BLK
    row, col = edge_index[0], edge_index[1]

    z3d = _tc_dense1(N, D, H, K1)(x, W1)

    partd = _make_deg(N, E)(row)
    dis = _tc_dis(N, nb)(partd, partd)

    prop_call = _make_prop(N, E, H)

    bk2 = jnp.zeros((N, H), jnp.float32)
    y = _tc_scale(N, H, K1 - 1)(z3d, dis)
    prev = z3d[K1 - 1]
    for k in range(K1 - 2, 0, -1):
        p = prop_call(y, row, col)
        bk, y = _tc_combine(N, H, nb, k)(z3d, p, p, bk2, dis)
        bk2 = prev
        prev = bk
    p = prop_call(y, row, col)
    return _tc_final(N, H, C, nb)(
        z3d, p, p, bk2, dis, b1.reshape(1, H), W2[0], b2.reshape(1, C))


# speedup vs baseline: 1.1255x; 1.0203x over previous
import jax
import jax.numpy as jnp
from jax import lax
from jax.experimental import pallas as pl
from jax.experimental.pallas import tpu as pltpu
from jax.experimental.pallas import tpu_sc as plsc

_NC = 2
_NS = 16
_NW = _NC * _NS

_SB = 128
_NSUB = 5
_GB = _SB * _NSUB
_WB = 624

_PARAMS = pltpu.CompilerParams(use_tc_tiling_on_sc=False)


def _mesh():
    return plsc.VectorSubcoreMesh(core_axis_name="c", subcore_axis_name="s")


def _slabs(N):
    out = [(0, _WB)]
    tail0 = _NS * _WB
    if tail0 < N:
        out.append((tail0, N - tail0))
    return out


def _make_prop(N, E, H):
    assert E % _GB == 0
    ncht = E // _GB
    nch = (ncht + _NW - 1) // _NW
    assert _NS * _WB <= N <= _NS * _WB + _WB and (N - _NS * _WB) % 8 == 0

    nbuf = 2
    nu = (nch + nbuf - 1) // nbuf

    def body(y_hbm, row_hbm, col_hbm, out_hbm, gidx, sidx, rows_v, acc_sh,
             sem, sems):
        ci = lax.axis_index("c")
        si = lax.axis_index("s")
        wid = si * _NC + ci

        def zrow(i, _):
            for j in range(H // 16):
                rows_v[0][i, pl.ds(j * 16, 16)] = jnp.zeros((16,), jnp.float32)
            return _
        lax.fori_loop(0, _GB, zrow, None)
        for base, n in _slabs(N):
            if base > 0:
                @pl.when(si == 0)
                def _():
                    pltpu.sync_copy(rows_v[0].at[pl.ds(0, n)],
                                    acc_sh.at[pl.ds(base, n)])
            else:
                done = 0
                while done < _WB:
                    m = min(_GB, _WB - done)
                    off = pl.multiple_of(si * _WB + done, 8)
                    pltpu.sync_copy(rows_v[0].at[pl.ds(0, m)],
                                    acc_sh.at[pl.ds(off, m)])
                    done += m
        plsc.subcore_barrier()

        def drain(b, pred):
            @pl.when(pred)
            def _():
                for j in range(_NSUB):
                    pltpu.make_async_copy(
                        rows_v[b].at[pl.ds(j * _SB, _SB)],
                        acc_sh.at[sidx[b][j]], sems[b]).wait()

        def step(u, _):
            for b in range(nbuf):
                c = (u * nbuf + b) * _NW + wid
                cprev = c - nbuf * _NW
                drain(b, (u > 0) & (cprev < ncht))

                @pl.when(c < ncht)
                def _():
                    eoff = pl.multiple_of(c * _GB, 8)
                    hs = [pltpu.async_copy(
                              row_hbm.at[pl.ds(eoff + j * _SB, _SB)],
                              gidx[b][j], sem)
                          for j in range(_NSUB)]
                    hs += [pltpu.async_copy(
                               col_hbm.at[pl.ds(eoff + j * _SB, _SB)],
                               sidx[b][j], sem)
                           for j in range(_NSUB)]
                    for h in hs:
                        h.wait()
                    gs = [pltpu.async_copy(y_hbm.at[gidx[b][j]],
                                           rows_v[b].at[pl.ds(j * _SB, _SB)],
                                           sem)
                          for j in range(_NSUB)]
                    for j in range(_NSUB):
                        gs[j].wait()
                        pltpu.async_copy(rows_v[b].at[pl.ds(j * _SB, _SB)],
                                         acc_sh.at[sidx[b][j]], sems[b],
                                         add=True)
            return _
        lax.fori_loop(0, nu, step, None)
        for b in range(nbuf):
            clast = ((nu - 1) * nbuf + b) * _NW + wid
            drain(b, clast < ncht)
        plsc.subcore_barrier()

        for base, n in _slabs(N):
            if base > 0:
                @pl.when(si == 0)
                def _():
                    pltpu.sync_copy(acc_sh.at[pl.ds(base, n)],
                                    rows_v[0].at[pl.ds(0, n)])
                    pltpu.sync_copy(rows_v[0].at[pl.ds(0, n)],
                                    out_hbm.at[pl.ds(ci * N + base, n)])
            else:
                done = 0
                while done < _WB:
                    m = min(_GB, _WB - done)
                    off = pl.multiple_of(si * _WB + done, 8)
                    pltpu.sync_copy(acc_sh.at[pl.ds(off, m)],
                                    rows_v[0].at[pl.ds(0, m)])
                    pltpu.sync_copy(rows_v[0].at[pl.ds(0, m)],
                                    out_hbm.at[pl.ds(ci * N + off, m)])
                    done += m

    return pl.kernel(
        body,
        out_type=jax.ShapeDtypeStruct((_NC * N, H), jnp.float32),
        mesh=_mesh(),
        compiler_params=_PARAMS,
        scratch_types=[
            [[pltpu.VMEM((_SB,), jnp.int32) for _ in range(_NSUB)]
             for _ in range(nbuf)],
            [[pltpu.VMEM((_SB,), jnp.int32) for _ in range(_NSUB)]
             for _ in range(nbuf)],
            [pltpu.VMEM((_GB, H), jnp.float32) for _ in range(nbuf)],
            pltpu.VMEM_SHARED((N, H), jnp.float32),
            pltpu.SemaphoreType.DMA,
            [pltpu.SemaphoreType.DMA for _ in range(nbuf)],
        ],
    )


def _make_deg(N, E):
    H = 16
    assert E % _GB == 0
    ncht = E // _GB
    nch = (ncht + _NW - 1) // _NW

    nbuf = 2
    nu = (nch + nbuf - 1) // nbuf

    def body(row_hbm, out_hbm, sidx, buf_v, acc_sh, sem, sems):
        ci = lax.axis_index("c")
        si = lax.axis_index("s")
        wid = si * _NC + ci

        def zrow(i, _):
            buf_v[i, pl.ds(0, 16)] = jnp.zeros((16,), jnp.float32)
            return _
        lax.fori_loop(0, _WB, zrow, None)
        for base, n in _slabs(N):
            if base > 0:
                @pl.when(si == 0)
                def _():
                    pltpu.sync_copy(buf_v.at[pl.ds(0, n)],
                                    acc_sh.at[pl.ds(base, n)])
            else:
                off = pl.multiple_of(si * _WB, 8)
                pltpu.sync_copy(buf_v.at[pl.ds(0, _WB)],
                                acc_sh.at[pl.ds(off, _WB)])

        def onerow(i, _):
            buf_v[i, pl.ds(0, 16)] = jnp.ones((16,), jnp.float32)
            return _
        lax.fori_loop(0, _SB, onerow, None)
        plsc.subcore_barrier()

        def drain(b, pred):
            @pl.when(pred)
            def _():
                for j in range(_NSUB):
                    pltpu.make_async_copy(buf_v.at[pl.ds(0, _SB)],
                                          acc_sh.at[sidx[b][j]],
                                          sems[b]).wait()

        def step(u, _):
            for b in range(nbuf):
                c = (u * nbuf + b) * _NW + wid
                cprev = c - nbuf * _NW
                drain(b, (u > 0) & (cprev < ncht))

                @pl.when(c < ncht)
                def _():
                    eoff = pl.multiple_of(c * _GB, 8)
                    hs = [pltpu.async_copy(
                              row_hbm.at[pl.ds(eoff + j * _SB, _SB)],
                              sidx[b][j], sem)
                          for j in range(_NSUB)]
                    for j in range(_NSUB):
                        hs[j].wait()
                        pltpu.async_copy(buf_v.at[pl.ds(0, _SB)],
                                         acc_sh.at[sidx[b][j]], sems[b],
                                         add=True)
            return _
        lax.fori_loop(0, nu, step, None)
        for b in range(nbuf):
            clast = ((nu - 1) * nbuf + b) * _NW + wid
            drain(b, clast < ncht)
        plsc.subcore_barrier()

        for base, n in _slabs(N):
            if base > 0:
                @pl.when(si == 0)
                def _():
                    pltpu.sync_copy(acc_sh.at[pl.ds(base, n)],
                                    buf_v.at[pl.ds(0, n)])
                    pltpu.sync_copy(buf_v.at[pl.ds(0, n)],
                                    out_hbm.at[pl.ds(ci * N + base, n)])
            else:
                off = pl.multiple_of(si * _WB, 8)
                pltpu.sync_copy(acc_sh.at[pl.ds(off, _WB)],
                                buf_v.at[pl.ds(0, _WB)])
                pltpu.sync_copy(buf_v.at[pl.ds(0, _WB)],
                                out_hbm.at[pl.ds(ci * N + off, _WB)])

    return pl.kernel(
        body,
        out_type=jax.ShapeDtypeStruct((_NC * N, H), jnp.float32),
        mesh=_mesh(),
        compiler_params=_PARAMS,
        scratch_types=[
            [[pltpu.VMEM((_SB,), jnp.int32) for _ in range(_NSUB)]
             for _ in range(nbuf)],
            pltpu.VMEM((_WB, H), jnp.float32),
            pltpu.VMEM_SHARED((N, H), jnp.float32),
            pltpu.SemaphoreType.DMA,
            [pltpu.SemaphoreType.DMA for _ in range(nbuf)],
        ],
    )


_BLK = 1000


def _tc_dense1(N, D, H, K1):
    def body(x_ref, w_ref, d_ref, o_ref, y_ref):
        for k in range(K1):
            o_ref[k] = jnp.dot(x_ref[...], w_ref[k],
                               preferred_element_type=jnp.float32)
        y_ref[...] = d_ref[...] * o_ref[K1 - 1]
    return pl.pallas_call(
        body,
        grid=(N // _BLK,),
        in_specs=[pl.BlockSpec((_BLK, D), lambda i: (i, 0)),
                  pl.BlockSpec((K1, D, H), lambda i: (0, 0, 0)),
                  pl.BlockSpec((_BLK, 1), lambda i: (i, 0))],
        out_specs=[pl.BlockSpec((K1, _BLK, H), lambda i: (0, i, 0)),
                   pl.BlockSpec((_BLK, H), lambda i: (i, 0))],
        out_shape=[jax.ShapeDtypeStruct((K1, N, H), jnp.float32),
                   jax.ShapeDtypeStruct((N, H), jnp.float32)],
    )


def _tc_dis(N, nb):
    def body(pt_ref, pb_ref, o_ref):
        deg = pt_ref[...][:, :1] + pb_ref[...][:, :1]
        o_ref[...] = jnp.where(deg > 0, lax.rsqrt(deg), 0.0)
    return pl.pallas_call(
        body,
        grid=(N // _BLK,),
        in_specs=[pl.BlockSpec((_BLK, 16), lambda i: (i, 0)),
                  pl.BlockSpec((_BLK, 16), lambda i: (i + nb, 0))],
        out_specs=pl.BlockSpec((_BLK, 1), lambda i: (i, 0)),
        out_shape=jax.ShapeDtypeStruct((N, 1), jnp.float32),
    )


def _tc_combine(N, H, nb, kcol):
    def body(z_ref, pt_ref, pb_ref, b2_ref, d_ref, obk_ref, oy_ref):
        u = d_ref[...] * (pt_ref[...] + pb_ref[...])
        bk = z_ref[0] - 2.0 * u - b2_ref[...]
        obk_ref[...] = bk
        oy_ref[...] = d_ref[...] * bk
    return pl.pallas_call(
        body,
        grid=(N // _BLK,),
        in_specs=[pl.BlockSpec((1, _BLK, H), lambda i: (kcol, i, 0)),
                  pl.BlockSpec((_BLK, H), lambda i: (i, 0)),
                  pl.BlockSpec((_BLK, H), lambda i: (i + nb, 0)),
                  pl.BlockSpec((_BLK, H), lambda i: (i, 0)),
                  pl.BlockSpec((_BLK, 1), lambda i: (i, 0))],
        out_specs=[pl.BlockSpec((_BLK, H), lambda i: (i, 0)),
                   pl.BlockSpec((_BLK, H), lambda i: (i, 0))],
        out_shape=[jax.ShapeDtypeStruct((N, H), jnp.float32),
                   jax.ShapeDtypeStruct((N, H), jnp.float32)],
    )


def _tc_final(N, H, C, nb):
    def body(z_ref, pt_ref, pb_ref, b2k_ref, d_ref, b1_ref, w2_ref, bb_ref,
             o_ref):
        hpre = (z_ref[0] - d_ref[...] * (pt_ref[...] + pb_ref[...])
                - b2k_ref[...] + b1_ref[...])
        h = jnp.maximum(hpre, 0.0)
        logits = jnp.dot(h, w2_ref[...],
                         preferred_element_type=jnp.float32) + bb_ref[...]
        m = jnp.max(logits, axis=1, keepdims=True)
        e = jnp.exp(logits - m)
        o_ref[...] = e / jnp.sum(e, axis=1, keepdims=True)
    return pl.pallas_call(
        body,
        grid=(N // _BLK,),
        in_specs=[pl.BlockSpec((1, _BLK, H), lambda i: (0, i, 0)),
                  pl.BlockSpec((_BLK, H), lambda i: (i, 0)),
                  pl.BlockSpec((_BLK, H), lambda i: (i + nb, 0)),
                  pl.BlockSpec((_BLK, H), lambda i: (i, 0)),
                  pl.BlockSpec((_BLK, 1), lambda i: (i, 0)),
                  pl.BlockSpec((1, H), lambda i: (0, 0)),
                  pl.BlockSpec((H, C), lambda i: (0, 0)),
                  pl.BlockSpec((1, C), lambda i: (0, 0))],
        out_specs=pl.BlockSpec((_BLK, C), lambda i: (i, 0)),
        out_shape=jax.ShapeDtypeStruct((N, C), jnp.float32),
    )


def kernel(x, edge_index, W1, b1, W2, b2):
    N, D = x.shape
    E = edge_index.shape[1]
    K1, _, H = W1.shape
    C = W2.shape[2]
    KH = K1 * H
    nb = N // _BLK
    row, col = edge_index[0], edge_index[1]

    partd = _make_deg(N, E)(row)
    dis = _tc_dis(N, nb)(partd, partd)

    z3d, y = _tc_dense1(N, D, H, K1)(x, W1, dis)

    prop_call = _make_prop(N, E, H)

    bk2 = jnp.zeros((N, H), jnp.float32)
    prev = z3d[K1 - 1]
    for k in range(K1 - 2, 0, -1):
        p = prop_call(y, row, col)
        bk, y = _tc_combine(N, H, nb, k)(z3d, p, p, bk2, dis)
        bk2 = prev
        prev = bk
    p = prop_call(y, row, col)
    return _tc_final(N, H, C, nb)(
        z3d, p, p, bk2, dis, b1.reshape(1, H), W2[0], b2.reshape(1, C))


# speedup vs baseline: 1.2291x; 1.0920x over previous
import jax
import jax.numpy as jnp
from jax import lax
from jax.experimental import pallas as pl
from jax.experimental.pallas import tpu as pltpu
from jax.experimental.pallas import tpu_sc as plsc

_NC = 2
_NS = 16
_NW = _NC * _NS

_SB = 128
_NSUB = 5
_GB = _SB * _NSUB
_WB = 624

_PARAMS = pltpu.CompilerParams(use_tc_tiling_on_sc=False)


def _mesh():
    return plsc.VectorSubcoreMesh(core_axis_name="c", subcore_axis_name="s")


def _slabs(N):
    out = [(0, _WB)]
    tail0 = _NS * _WB
    if tail0 < N:
        out.append((tail0, N - tail0))
    return out


def _make_prop(N, E, H):
    assert E % _GB == 0
    ncht = E // _GB
    nch = (ncht + _NW - 1) // _NW
    assert _NS * _WB <= N <= _NS * _WB + _WB and (N - _NS * _WB) % 8 == 0

    nbuf = 2
    nu = (nch + nbuf - 1) // nbuf

    def body(y_hbm, row_hbm, col_hbm, out_hbm, gidx, sidx, rows_v, acc_sh,
             sem, sems, gsems):
        ci = lax.axis_index("c")
        si = lax.axis_index("s")
        wid = si * _NC + ci

        def zrow(i, _):
            for j in range(H // 16):
                rows_v[0][i, pl.ds(j * 16, 16)] = jnp.zeros((16,), jnp.float32)
            return _
        lax.fori_loop(0, _GB, zrow, None)
        for base, n in _slabs(N):
            if base > 0:
                @pl.when(si == 0)
                def _():
                    pltpu.sync_copy(rows_v[0].at[pl.ds(0, n)],
                                    acc_sh.at[pl.ds(base, n)])
            else:
                done = 0
                while done < _WB:
                    m = min(_GB, _WB - done)
                    off = pl.multiple_of(si * _WB + done, 8)
                    pltpu.sync_copy(rows_v[0].at[pl.ds(0, m)],
                                    acc_sh.at[pl.ds(off, m)])
                    done += m
        plsc.subcore_barrier()

        def drain(b, pred):
            @pl.when(pred)
            def _():
                for j in range(_NSUB):
                    pltpu.make_async_copy(
                        rows_v[b].at[pl.ds(j * _SB, _SB)],
                        acc_sh.at[sidx[b][j]], sems[b]).wait()

        def stage_a(b, c):
            @pl.when(c < ncht)
            def _():
                eoff = pl.multiple_of(c * _GB, 8)
                hs = [pltpu.async_copy(
                          row_hbm.at[pl.ds(eoff + j * _SB, _SB)],
                          gidx[b][j], sem)
                      for j in range(_NSUB)]
                hs += [pltpu.async_copy(
                           col_hbm.at[pl.ds(eoff + j * _SB, _SB)],
                           sidx[b][j], sem)
                       for j in range(_NSUB)]
                for h in hs:
                    h.wait()
                for j in range(_NSUB):
                    pltpu.async_copy(y_hbm.at[gidx[b][j]],
                                     rows_v[b].at[pl.ds(j * _SB, _SB)],
                                     gsems[b])
            return _

        def stage_b(b, c, lo_ok):
            pred = (c < ncht) & lo_ok if lo_ok is not None else c < ncht

            @pl.when(pred)
            def _():
                for j in range(_NSUB):
                    pltpu.make_async_copy(
                        y_hbm.at[gidx[b][j]],
                        rows_v[b].at[pl.ds(j * _SB, _SB)], gsems[b]).wait()
                    pltpu.async_copy(rows_v[b].at[pl.ds(j * _SB, _SB)],
                                     acc_sh.at[sidx[b][j]], sems[b],
                                     add=True)

        def step(u, _):
            for b in range(nbuf):
                c = (u * nbuf + b) * _NW + wid
                drain(b, (u > 0) & (c - nbuf * _NW < ncht))
                stage_a(b, c)
                cb = c - _NW
                stage_b(b - 1 if b > 0 else nbuf - 1, cb, cb >= 0)
            return _
        lax.fori_loop(0, nu, step, None)
        clastslot = ((nu - 1) * nbuf + (nbuf - 1)) * _NW + wid
        stage_b(nbuf - 1, clastslot, None)
        for b in range(nbuf):
            clast = ((nu - 1) * nbuf + b) * _NW + wid
            drain(b, clast < ncht)
        plsc.subcore_barrier()

        for base, n in _slabs(N):
            if base > 0:
                @pl.when(si == 0)
                def _():
                    pltpu.sync_copy(acc_sh.at[pl.ds(base, n)],
                                    rows_v[0].at[pl.ds(0, n)])
                    pltpu.sync_copy(rows_v[0].at[pl.ds(0, n)],
                                    out_hbm.at[pl.ds(ci * N + base, n)])
            else:
                done = 0
                while done < _WB:
                    m = min(_GB, _WB - done)
                    off = pl.multiple_of(si * _WB + done, 8)
                    pltpu.sync_copy(acc_sh.at[pl.ds(off, m)],
                                    rows_v[0].at[pl.ds(0, m)])
                    pltpu.sync_copy(rows_v[0].at[pl.ds(0, m)],
                                    out_hbm.at[pl.ds(ci * N + off, m)])
                    done += m

    return pl.kernel(
        body,
        out_type=jax.ShapeDtypeStruct((_NC * N, H), jnp.float32),
        mesh=_mesh(),
        compiler_params=_PARAMS,
        scratch_types=[
            [[pltpu.VMEM((_SB,), jnp.int32) for _ in range(_NSUB)]
             for _ in range(nbuf)],
            [[pltpu.VMEM((_SB,), jnp.int32) for _ in range(_NSUB)]
             for _ in range(nbuf)],
            [pltpu.VMEM((_GB, H), jnp.float32) for _ in range(nbuf)],
            pltpu.VMEM_SHARED((N, H), jnp.float32),
            pltpu.SemaphoreType.DMA,
            [pltpu.SemaphoreType.DMA for _ in range(nbuf)],
            [pltpu.SemaphoreType.DMA for _ in range(nbuf)],
        ],
    )


def _make_deg(N, E):
    H = 16
    assert E % _GB == 0
    ncht = E // _GB
    nch = (ncht + _NW - 1) // _NW

    nbuf = 2
    nu = (nch + nbuf - 1) // nbuf

    def body(row_hbm, out_hbm, sidx, buf_v, acc_sh, sem, sems):
        ci = lax.axis_index("c")
        si = lax.axis_index("s")
        wid = si * _NC + ci

        def zrow(i, _):
            buf_v[i, pl.ds(0, 16)] = jnp.zeros((16,), jnp.float32)
            return _
        lax.fori_loop(0, _WB, zrow, None)
        for base, n in _slabs(N):
            if base > 0:
                @pl.when(si == 0)
                def _():
                    pltpu.sync_copy(buf_v.at[pl.ds(0, n)],
                                    acc_sh.at[pl.ds(base, n)])
            else:
                off = pl.multiple_of(si * _WB, 8)
                pltpu.sync_copy(buf_v.at[pl.ds(0, _WB)],
                                acc_sh.at[pl.ds(off, _WB)])

        def onerow(i, _):
            buf_v[i, pl.ds(0, 16)] = jnp.ones((16,), jnp.float32)
            return _
        lax.fori_loop(0, _SB, onerow, None)
        plsc.subcore_barrier()

        def drain(b, pred):
            @pl.when(pred)
            def _():
                for j in range(_NSUB):
                    pltpu.make_async_copy(buf_v.at[pl.ds(0, _SB)],
                                          acc_sh.at[sidx[b][j]],
                                          sems[b]).wait()

        def step(u, _):
            for b in range(nbuf):
                c = (u * nbuf + b) * _NW + wid
                cprev = c - nbuf * _NW
                drain(b, (u > 0) & (cprev < ncht))

                @pl.when(c < ncht)
                def _():
                    eoff = pl.multiple_of(c * _GB, 8)
                    hs = [pltpu.async_copy(
                              row_hbm.at[pl.ds(eoff + j * _SB, _SB)],
                              sidx[b][j], sem)
                          for j in range(_NSUB)]
                    for j in range(_NSUB):
                        hs[j].wait()
                        pltpu.async_copy(buf_v.at[pl.ds(0, _SB)],
                                         acc_sh.at[sidx[b][j]], sems[b],
                                         add=True)
            return _
        lax.fori_loop(0, nu, step, None)
        for b in range(nbuf):
            clast = ((nu - 1) * nbuf + b) * _NW + wid
            drain(b, clast < ncht)
        plsc.subcore_barrier()

        for base, n in _slabs(N):
            if base > 0:
                @pl.when(si == 0)
                def _():
                    pltpu.sync_copy(acc_sh.at[pl.ds(base, n)],
                                    buf_v.at[pl.ds(0, n)])
                    pltpu.sync_copy(buf_v.at[pl.ds(0, n)],
                                    out_hbm.at[pl.ds(ci * N + base, n)])
            else:
                off = pl.multiple_of(si * _WB, 8)
                pltpu.sync_copy(acc_sh.at[pl.ds(off, _WB)],
                                buf_v.at[pl.ds(0, _WB)])
                pltpu.sync_copy(buf_v.at[pl.ds(0, _WB)],
                                out_hbm.at[pl.ds(ci * N + off, _WB)])

    return pl.kernel(
        body,
        out_type=jax.ShapeDtypeStruct((_NC * N, H), jnp.float32),
        mesh=_mesh(),
        compiler_params=_PARAMS,
        scratch_types=[
            [[pltpu.VMEM((_SB,), jnp.int32) for _ in range(_NSUB)]
             for _ in range(nbuf)],
            pltpu.VMEM((_WB, H), jnp.float32),
            pltpu.VMEM_SHARED((N, H), jnp.float32),
            pltpu.SemaphoreType.DMA,
            [pltpu.SemaphoreType.DMA for _ in range(nbuf)],
        ],
    )


_BLK = 1000


def _tc_dense1(N, D, H, K1):
    def body(x_ref, w_ref, d_ref, o_ref, y_ref):
        for k in range(K1):
            o_ref[k] = jnp.dot(x_ref[...], w_ref[k],
                               preferred_element_type=jnp.float32)
        y_ref[...] = d_ref[...] * o_ref[K1 - 1]
    return pl.pallas_call(
        body,
        grid=(N // _BLK,),
        in_specs=[pl.BlockSpec((_BLK, D), lambda i: (i, 0)),
                  pl.BlockSpec((K1, D, H), lambda i: (0, 0, 0)),
                  pl.BlockSpec((_BLK, 1), lambda i: (i, 0))],
        out_specs=[pl.BlockSpec((K1, _BLK, H), lambda i: (0, i, 0)),
                   pl.BlockSpec((_BLK, H), lambda i: (i, 0))],
        out_shape=[jax.ShapeDtypeStruct((K1, N, H), jnp.float32),
                   jax.ShapeDtypeStruct((N, H), jnp.float32)],
    )


def _tc_dis(N, nb):
    def body(pt_ref, pb_ref, o_ref):
        deg = pt_ref[...][:, :1] + pb_ref[...][:, :1]
        o_ref[...] = jnp.where(deg > 0, lax.rsqrt(deg), 0.0)
    return pl.pallas_call(
        body,
        grid=(N // _BLK,),
        in_specs=[pl.BlockSpec((_BLK, 16), lambda i: (i, 0)),
                  pl.BlockSpec((_BLK, 16), lambda i: (i + nb, 0))],
        out_specs=pl.BlockSpec((_BLK, 1), lambda i: (i, 0)),
        out_shape=jax.ShapeDtypeStruct((N, 1), jnp.float32),
    )


def _tc_combine(N, H, nb, kcol):
    def body(z_ref, pt_ref, pb_ref, b2_ref, d_ref, obk_ref, oy_ref):
        u = d_ref[...] * (pt_ref[...] + pb_ref[...])
        bk = z_ref[0] - 2.0 * u - b2_ref[...]
        obk_ref[...] = bk
        oy_ref[...] = d_ref[...] * bk
    return pl.pallas_call(
        body,
        grid=(N // _BLK,),
        in_specs=[pl.BlockSpec((1, _BLK, H), lambda i: (kcol, i, 0)),
                  pl.BlockSpec((_BLK, H), lambda i: (i, 0)),
                  pl.BlockSpec((_BLK, H), lambda i: (i + nb, 0)),
                  pl.BlockSpec((_BLK, H), lambda i: (i, 0)),
                  pl.BlockSpec((_BLK, 1), lambda i: (i, 0))],
        out_specs=[pl.BlockSpec((_BLK, H), lambda i: (i, 0)),
                   pl.BlockSpec((_BLK, H), lambda i: (i, 0))],
        out_shape=[jax.ShapeDtypeStruct((N, H), jnp.float32),
                   jax.ShapeDtypeStruct((N, H), jnp.float32)],
    )


def _tc_final(N, H, C, nb):
    def body(z_ref, pt_ref, pb_ref, b2k_ref, d_ref, b1_ref, w2_ref, bb_ref,
             o_ref):
        hpre = (z_ref[0] - d_ref[...] * (pt_ref[...] + pb_ref[...])
                - b2k_ref[...] + b1_ref[...])
        h = jnp.maximum(hpre, 0.0)
        logits = jnp.dot(h, w2_ref[...],
                         preferred_element_type=jnp.float32) + bb_ref[...]
        m = jnp.max(logits, axis=1, keepdims=True)
        e = jnp.exp(logits - m)
        o_ref[...] = e / jnp.sum(e, axis=1, keepdims=True)
    return pl.pallas_call(
        body,
        grid=(N // _BLK,),
        in_specs=[pl.BlockSpec((1, _BLK, H), lambda i: (0, i, 0)),
                  pl.BlockSpec((_BLK, H), lambda i: (i, 0)),
                  pl.BlockSpec((_BLK, H), lambda i: (i + nb, 0)),
                  pl.BlockSpec((_BLK, H), lambda i: (i, 0)),
                  pl.BlockSpec((_BLK, 1), lambda i: (i, 0)),
                  pl.BlockSpec((1, H), lambda i: (0, 0)),
                  pl.BlockSpec((H, C), lambda i: (0, 0)),
                  pl.BlockSpec((1, C), lambda i: (0, 0))],
        out_specs=pl.BlockSpec((_BLK, C), lambda i: (i, 0)),
        out_shape=jax.ShapeDtypeStruct((N, C), jnp.float32),
    )


def kernel(x, edge_index, W1, b1, W2, b2):
    N, D = x.shape
    E = edge_index.shape[1]
    K1, _, H = W1.shape
    C = W2.shape[2]
    KH = K1 * H
    nb = N // _BLK
    row, col = edge_index[0], edge_index[1]

    partd = _make_deg(N, E)(row)
    dis = _tc_dis(N, nb)(partd, partd)

    z3d, y = _tc_dense1(N, D, H, K1)(x, W1, dis)

    prop_call = _make_prop(N, E, H)

    bk2 = jnp.zeros((N, H), jnp.float32)
    prev = z3d[K1 - 1]
    for k in range(K1 - 2, 0, -1):
        p = prop_call(y, row, col)
        bk, y = _tc_combine(N, H, nb, k)(z3d, p, p, bk2, dis)
        bk2 = prev
        prev = bk
    p = prop_call(y, row, col)
    return _tc_final(N, H, C, nb)(
        z3d, p, p, bk2, dis, b1.reshape(1, H), W2[0], b2.reshape(1, C))


# speedup vs baseline: 1.2478x; 1.0152x over previous
import jax
import jax.numpy as jnp
from jax import lax
from jax.experimental import pallas as pl
from jax.experimental.pallas import tpu as pltpu
from jax.experimental.pallas import tpu_sc as plsc

_NC = 2
_NS = 16
_NW = _NC * _NS

_SB = 128
_NSUB = 5
_GB = _SB * _NSUB
_WB = 624

_PARAMS = pltpu.CompilerParams(use_tc_tiling_on_sc=False)


def _mesh():
    return plsc.VectorSubcoreMesh(core_axis_name="c", subcore_axis_name="s")


def _slabs(N):
    out = [(0, _WB)]
    tail0 = _NS * _WB
    if tail0 < N:
        out.append((tail0, N - tail0))
    return out


def _make_prop(N, E, H):
    assert E % _GB == 0
    ncht = E // _GB
    nch = (ncht + _NW - 1) // _NW
    assert _NS * _WB <= N <= _NS * _WB + _WB and (N - _NS * _WB) % 8 == 0

    nbuf = 2
    nu = (nch + nbuf - 1) // nbuf

    def body(y_hbm, row_hbm, col_hbm, out_hbm, gidx, sidx, rows_v, acc_sh,
             sem, sems, gsems):
        ci = lax.axis_index("c")
        si = lax.axis_index("s")
        wid = si * _NC + ci

        def zrow(i, _):
            for j in range(H // 16):
                rows_v[0][i, pl.ds(j * 16, 16)] = jnp.zeros((16,), jnp.float32)
            return _
        lax.fori_loop(0, _GB, zrow, None)
        for base, n in _slabs(N):
            if base > 0:
                @pl.when(si == 0)
                def _():
                    pltpu.sync_copy(rows_v[0].at[pl.ds(0, n)],
                                    acc_sh.at[pl.ds(base, n)])
            else:
                done = 0
                while done < _WB:
                    m = min(_GB, _WB - done)
                    off = pl.multiple_of(si * _WB + done, 8)
                    pltpu.sync_copy(rows_v[0].at[pl.ds(0, m)],
                                    acc_sh.at[pl.ds(off, m)])
                    done += m
        plsc.subcore_barrier()

        def drain(b, pred):
            @pl.when(pred)
            def _():
                for j in range(_NSUB):
                    pltpu.make_async_copy(
                        rows_v[b].at[pl.ds(j * _SB, _SB)],
                        acc_sh.at[sidx[b][j]], sems[b]).wait()

        def stage_a(b, c):
            @pl.when(c < ncht)
            def _():
                eoff = pl.multiple_of(c * _GB, 8)

                @pl.when(c - wid < nbuf * _NW)
                def _():
                    for j in range(_NSUB):
                        pltpu.async_copy(
                            row_hbm.at[pl.ds(eoff + j * _SB, _SB)],
                            gidx[b][j], sem)
                hs = [pltpu.async_copy(
                          col_hbm.at[pl.ds(eoff + j * _SB, _SB)],
                          sidx[b][j], sem)
                      for j in range(_NSUB)]
                for j in range(_NSUB):
                    pltpu.make_async_copy(
                        row_hbm.at[pl.ds(eoff + j * _SB, _SB)],
                        gidx[b][j], sem).wait()
                for h in hs:
                    h.wait()
                for j in range(_NSUB):
                    pltpu.async_copy(y_hbm.at[gidx[b][j]],
                                     rows_v[b].at[pl.ds(j * _SB, _SB)],
                                     gsems[b])
            return _

        def stage_b(b, c, lo_ok):
            pred = (c < ncht) & lo_ok if lo_ok is not None else c < ncht

            @pl.when(pred)
            def _():
                for j in range(_NSUB):
                    pltpu.make_async_copy(
                        y_hbm.at[gidx[b][j]],
                        rows_v[b].at[pl.ds(j * _SB, _SB)], gsems[b]).wait()
                    pltpu.async_copy(rows_v[b].at[pl.ds(j * _SB, _SB)],
                                     acc_sh.at[sidx[b][j]], sems[b],
                                     add=True)
                cn = c + nbuf * _NW

                @pl.when(cn < ncht)
                def _():
                    eoffn = pl.multiple_of(cn * _GB, 8)
                    for j in range(_NSUB):
                        pltpu.async_copy(
                            row_hbm.at[pl.ds(eoffn + j * _SB, _SB)],
                            gidx[b][j], sem)

        def step(u, _):
            for b in range(nbuf):
                c = (u * nbuf + b) * _NW + wid
                drain(b, (u > 0) & (c - nbuf * _NW < ncht))
                stage_a(b, c)
                cb = c - _NW
                stage_b(b - 1 if b > 0 else nbuf - 1, cb, cb >= 0)
            return _
        lax.fori_loop(0, nu, step, None)
        clastslot = ((nu - 1) * nbuf + (nbuf - 1)) * _NW + wid
        stage_b(nbuf - 1, clastslot, None)
        for b in range(nbuf):
            clast = ((nu - 1) * nbuf + b) * _NW + wid
            drain(b, clast < ncht)
        plsc.subcore_barrier()

        for base, n in _slabs(N):
            if base > 0:
                @pl.when(si == 0)
                def _():
                    pltpu.sync_copy(acc_sh.at[pl.ds(base, n)],
                                    rows_v[0].at[pl.ds(0, n)])
                    pltpu.sync_copy(rows_v[0].at[pl.ds(0, n)],
                                    out_hbm.at[pl.ds(ci * N + base, n)])
            else:
                done = 0
                while done < _WB:
                    m = min(_GB, _WB - done)
                    off = pl.multiple_of(si * _WB + done, 8)
                    pltpu.sync_copy(acc_sh.at[pl.ds(off, m)],
                                    rows_v[0].at[pl.ds(0, m)])
                    pltpu.sync_copy(rows_v[0].at[pl.ds(0, m)],
                                    out_hbm.at[pl.ds(ci * N + off, m)])
                    done += m

    return pl.kernel(
        body,
        out_type=jax.ShapeDtypeStruct((_NC * N, H), jnp.float32),
        mesh=_mesh(),
        compiler_params=_PARAMS,
        scratch_types=[
            [[pltpu.VMEM((_SB,), jnp.int32) for _ in range(_NSUB)]
             for _ in range(nbuf)],
            [[pltpu.VMEM((_SB,), jnp.int32) for _ in range(_NSUB)]
             for _ in range(nbuf)],
            [pltpu.VMEM((_GB, H), jnp.float32) for _ in range(nbuf)],
            pltpu.VMEM_SHARED((N, H), jnp.float32),
            pltpu.SemaphoreType.DMA,
            [pltpu.SemaphoreType.DMA for _ in range(nbuf)],
            [pltpu.SemaphoreType.DMA for _ in range(nbuf)],
        ],
    )


def _make_deg(N, E):
    H = 16
    assert E % _GB == 0
    ncht = E // _GB
    nch = (ncht + _NW - 1) // _NW

    nbuf = 2
    nu = (nch + nbuf - 1) // nbuf

    def body(row_hbm, out_hbm, sidx, buf_v, acc_sh, sem, sems):
        ci = lax.axis_index("c")
        si = lax.axis_index("s")
        wid = si * _NC + ci

        def zrow(i, _):
            buf_v[i, pl.ds(0, 16)] = jnp.zeros((16,), jnp.float32)
            return _
        lax.fori_loop(0, _WB, zrow, None)
        for base, n in _slabs(N):
            if base > 0:
                @pl.when(si == 0)
                def _():
                    pltpu.sync_copy(buf_v.at[pl.ds(0, n)],
                                    acc_sh.at[pl.ds(base, n)])
            else:
                off = pl.multiple_of(si * _WB, 8)
                pltpu.sync_copy(buf_v.at[pl.ds(0, _WB)],
                                acc_sh.at[pl.ds(off, _WB)])

        def onerow(i, _):
            buf_v[i, pl.ds(0, 16)] = jnp.ones((16,), jnp.float32)
            return _
        lax.fori_loop(0, _SB, onerow, None)
        plsc.subcore_barrier()

        def drain(b, pred):
            @pl.when(pred)
            def _():
                for j in range(_NSUB):
                    pltpu.make_async_copy(buf_v.at[pl.ds(0, _SB)],
                                          acc_sh.at[sidx[b][j]],
                                          sems[b]).wait()

        def step(u, _):
            for b in range(nbuf):
                c = (u * nbuf + b) * _NW + wid
                cprev = c - nbuf * _NW
                drain(b, (u > 0) & (cprev < ncht))

                @pl.when(c < ncht)
                def _():
                    eoff = pl.multiple_of(c * _GB, 8)
                    hs = [pltpu.async_copy(
                              row_hbm.at[pl.ds(eoff + j * _SB, _SB)],
                              sidx[b][j], sem)
                          for j in range(_NSUB)]
                    for j in range(_NSUB):
                        hs[j].wait()
                        pltpu.async_copy(buf_v.at[pl.ds(0, _SB)],
                                         acc_sh.at[sidx[b][j]], sems[b],
                                         add=True)
            return _
        lax.fori_loop(0, nu, step, None)
        for b in range(nbuf):
            clast = ((nu - 1) * nbuf + b) * _NW + wid
            drain(b, clast < ncht)
        plsc.subcore_barrier()

        for base, n in _slabs(N):
            if base > 0:
                @pl.when(si == 0)
                def _():
                    pltpu.sync_copy(acc_sh.at[pl.ds(base, n)],
                                    buf_v.at[pl.ds(0, n)])
                    pltpu.sync_copy(buf_v.at[pl.ds(0, n)],
                                    out_hbm.at[pl.ds(ci * N + base, n)])
            else:
                off = pl.multiple_of(si * _WB, 8)
                pltpu.sync_copy(acc_sh.at[pl.ds(off, _WB)],
                                buf_v.at[pl.ds(0, _WB)])
                pltpu.sync_copy(buf_v.at[pl.ds(0, _WB)],
                                out_hbm.at[pl.ds(ci * N + off, _WB)])

    return pl.kernel(
        body,
        out_type=jax.ShapeDtypeStruct((_NC * N, H), jnp.float32),
        mesh=_mesh(),
        compiler_params=_PARAMS,
        scratch_types=[
            [[pltpu.VMEM((_SB,), jnp.int32) for _ in range(_NSUB)]
             for _ in range(nbuf)],
            pltpu.VMEM((_WB, H), jnp.float32),
            pltpu.VMEM_SHARED((N, H), jnp.float32),
            pltpu.SemaphoreType.DMA,
            [pltpu.SemaphoreType.DMA for _ in range(nbuf)],
        ],
    )


_BLK = 1000


def _tc_dense1(N, D, H, K1, nb):
    def body(x_ref, w_ref, pt_ref, pb_ref, o_ref, y_ref, d_ref):
        deg = pt_ref[...][:, :1] + pb_ref[...][:, :1]
        d = jnp.where(deg > 0, lax.rsqrt(deg), 0.0)
        d_ref[...] = d
        for k in range(K1):
            o_ref[k] = jnp.dot(x_ref[...], w_ref[k],
                               preferred_element_type=jnp.float32)
        y_ref[...] = d * o_ref[K1 - 1]
    return pl.pallas_call(
        body,
        grid=(N // _BLK,),
        in_specs=[pl.BlockSpec((_BLK, D), lambda i: (i, 0)),
                  pl.BlockSpec((K1, D, H), lambda i: (0, 0, 0)),
                  pl.BlockSpec((_BLK, 16), lambda i: (i, 0)),
                  pl.BlockSpec((_BLK, 16), lambda i: (i + nb, 0))],
        out_specs=[pl.BlockSpec((K1, _BLK, H), lambda i: (0, i, 0)),
                   pl.BlockSpec((_BLK, H), lambda i: (i, 0)),
                   pl.BlockSpec((_BLK, 1), lambda i: (i, 0))],
        out_shape=[jax.ShapeDtypeStruct((K1, N, H), jnp.float32),
                   jax.ShapeDtypeStruct((N, H), jnp.float32),
                   jax.ShapeDtypeStruct((N, 1), jnp.float32)],
    )


def _tc_combine(N, H, nb, kcol):
    def body(z_ref, pt_ref, pb_ref, b2_ref, d_ref, obk_ref, oy_ref):
        u = d_ref[...] * (pt_ref[...] + pb_ref[...])
        bk = z_ref[0] - 2.0 * u - b2_ref[...]
        obk_ref[...] = bk
        oy_ref[...] = d_ref[...] * bk
    return pl.pallas_call(
        body,
        grid=(N // _BLK,),
        in_specs=[pl.BlockSpec((1, _BLK, H), lambda i: (kcol, i, 0)),
                  pl.BlockSpec((_BLK, H), lambda i: (i, 0)),
                  pl.BlockSpec((_BLK, H), lambda i: (i + nb, 0)),
                  pl.BlockSpec((_BLK, H), lambda i: (i, 0)),
                  pl.BlockSpec((_BLK, 1), lambda i: (i, 0))],
        out_specs=[pl.BlockSpec((_BLK, H), lambda i: (i, 0)),
                   pl.BlockSpec((_BLK, H), lambda i: (i, 0))],
        out_shape=[jax.ShapeDtypeStruct((N, H), jnp.float32),
                   jax.ShapeDtypeStruct((N, H), jnp.float32)],
    )


def _tc_final(N, H, C, nb):
    def body(z_ref, pt_ref, pb_ref, b2k_ref, d_ref, b1_ref, w2_ref, bb_ref,
             o_ref):
        hpre = (z_ref[0] - d_ref[...] * (pt_ref[...] + pb_ref[...])
                - b2k_ref[...] + b1_ref[...])
        h = jnp.maximum(hpre, 0.0)
        logits = jnp.dot(h, w2_ref[...],
                         preferred_element_type=jnp.float32) + bb_ref[...]
        m = jnp.max(logits, axis=1, keepdims=True)
        e = jnp.exp(logits - m)
        o_ref[...] = e / jnp.sum(e, axis=1, keepdims=True)
    return pl.pallas_call(
        body,
        grid=(N // _BLK,),
        in_specs=[pl.BlockSpec((1, _BLK, H), lambda i: (0, i, 0)),
                  pl.BlockSpec((_BLK, H), lambda i: (i, 0)),
                  pl.BlockSpec((_BLK, H), lambda i: (i + nb, 0)),
                  pl.BlockSpec((_BLK, H), lambda i: (i, 0)),
                  pl.BlockSpec((_BLK, 1), lambda i: (i, 0)),
                  pl.BlockSpec((1, H), lambda i: (0, 0)),
                  pl.BlockSpec((H, C), lambda i: (0, 0)),
                  pl.BlockSpec((1, C), lambda i: (0, 0))],
        out_specs=pl.BlockSpec((_BLK, C), lambda i: (i, 0)),
        out_shape=jax.ShapeDtypeStruct((N, C), jnp.float32),
    )


def kernel(x, edge_index, W1, b1, W2, b2):
    N, D = x.shape
    E = edge_index.shape[1]
    K1, _, H = W1.shape
    C = W2.shape[2]
    KH = K1 * H
    nb = N // _BLK
    row, col = edge_index[0], edge_index[1]

    partd = _make_deg(N, E)(row)

    z3d, y, dis = _tc_dense1(N, D, H, K1, nb)(x, W1, partd, partd)

    prop_call = _make_prop(N, E, H)

    bk2 = jnp.zeros((N, H), jnp.float32)
    prev = z3d[K1 - 1]
    for k in range(K1 - 2, 0, -1):
        p = prop_call(y, row, col)
        bk, y = _tc_combine(N, H, nb, k)(z3d, p, p, bk2, dis)
        bk2 = prev
        prev = bk
    p = prop_call(y, row, col)
    return _tc_final(N, H, C, nb)(
        z3d, p, p, bk2, dis, b1.reshape(1, H), W2[0], b2.reshape(1, C))


# speedup vs baseline: 1.2518x; 1.0032x over previous
import jax
import jax.numpy as jnp
from jax import lax
from jax.experimental import pallas as pl
from jax.experimental.pallas import tpu as pltpu
from jax.experimental.pallas import tpu_sc as plsc

_NC = 2
_NS = 16
_NW = _NC * _NS

_SB = 128
_NSUB = 5
_GB = _SB * _NSUB
_WB = 624

_PARAMS = pltpu.CompilerParams(use_tc_tiling_on_sc=False)


def _mesh():
    return plsc.VectorSubcoreMesh(core_axis_name="c", subcore_axis_name="s")


def _slabs(N):
    out = [(0, _WB)]
    tail0 = _NS * _WB
    if tail0 < N:
        out.append((tail0, N - tail0))
    return out


def _make_prop(N, E, H):
    assert E % _GB == 0
    ncht = E // _GB
    nch = (ncht + _NW - 1) // _NW
    assert _NS * _WB <= N <= _NS * _WB + _WB and (N - _NS * _WB) % 8 == 0

    nbuf = 2
    nu = (nch + nbuf - 1) // nbuf

    def body(y_hbm, row_hbm, col_hbm, out_hbm, gidx, sidx, rows_v, acc_sh,
             sem, sems, gsems):
        ci = lax.axis_index("c")
        si = lax.axis_index("s")
        wid = si * _NC + ci

        def zrow(i, _):
            for j in range(H // 16):
                rows_v[0][i, pl.ds(j * 16, 16)] = jnp.zeros((16,), jnp.float32)
            return _
        lax.fori_loop(0, _GB, zrow, None)
        for base, n in _slabs(N):
            if base > 0:
                @pl.when(si == 0)
                def _():
                    pltpu.sync_copy(rows_v[0].at[pl.ds(0, n)],
                                    acc_sh.at[pl.ds(base, n)])
            else:
                done = 0
                while done < _WB:
                    m = min(_GB, _WB - done)
                    off = pl.multiple_of(si * _WB + done, 8)
                    pltpu.sync_copy(rows_v[0].at[pl.ds(0, m)],
                                    acc_sh.at[pl.ds(off, m)])
                    done += m
        plsc.subcore_barrier()

        def drain(b, pred):
            @pl.when(pred)
            def _():
                for j in range(_NSUB):
                    pltpu.make_async_copy(
                        rows_v[b].at[pl.ds(j * _SB, _SB)],
                        acc_sh.at[sidx[b][j]], sems[b]).wait()

        def stage_a(b, c):
            @pl.when(c < ncht)
            def _():
                eoff = pl.multiple_of(c * _GB, 8)

                @pl.when(c - wid < nbuf * _NW)
                def _():
                    for j in range(_NSUB):
                        pltpu.async_copy(
                            row_hbm.at[pl.ds(eoff + j * _SB, _SB)],
                            gidx[b][j], sem)
                hs = [pltpu.async_copy(
                          col_hbm.at[pl.ds(eoff + j * _SB, _SB)],
                          sidx[b][j], sem)
                      for j in range(_NSUB)]
                for j in range(_NSUB):
                    pltpu.make_async_copy(
                        row_hbm.at[pl.ds(eoff + j * _SB, _SB)],
                        gidx[b][j], sem).wait()
                for h in hs:
                    h.wait()
                for j in range(_NSUB):
                    pltpu.async_copy(y_hbm.at[gidx[b][j]],
                                     rows_v[b].at[pl.ds(j * _SB, _SB)],
                                     gsems[b])
            return _

        def stage_b(b, c, lo_ok):
            pred = (c < ncht) & lo_ok if lo_ok is not None else c < ncht

            @pl.when(pred)
            def _():
                for j in range(_NSUB):
                    pltpu.make_async_copy(
                        y_hbm.at[gidx[b][j]],
                        rows_v[b].at[pl.ds(j * _SB, _SB)], gsems[b]).wait()
                    pltpu.async_copy(rows_v[b].at[pl.ds(j * _SB, _SB)],
                                     acc_sh.at[sidx[b][j]], sems[b],
                                     add=True)
                cn = c + nbuf * _NW

                @pl.when(cn < ncht)
                def _():
                    eoffn = pl.multiple_of(cn * _GB, 8)
                    for j in range(_NSUB):
                        pltpu.async_copy(
                            row_hbm.at[pl.ds(eoffn + j * _SB, _SB)],
                            gidx[b][j], sem)

        def step(u, _):
            for b in range(nbuf):
                c = (u * nbuf + b) * _NW + wid
                drain(b, (u > 0) & (c - nbuf * _NW < ncht))
                stage_a(b, c)
                cb = c - _NW
                stage_b(b - 1 if b > 0 else nbuf - 1, cb, cb >= 0)
            return _
        lax.fori_loop(0, nu, step, None)
        clastslot = ((nu - 1) * nbuf + (nbuf - 1)) * _NW + wid
        stage_b(nbuf - 1, clastslot, None)
        for b in range(nbuf):
            clast = ((nu - 1) * nbuf + b) * _NW + wid
            drain(b, clast < ncht)
        plsc.subcore_barrier()

        for base, n in _slabs(N):
            if base > 0:
                @pl.when(si == 0)
                def _():
                    pltpu.sync_copy(acc_sh.at[pl.ds(base, n)],
                                    rows_v[0].at[pl.ds(0, n)])
                    pltpu.sync_copy(rows_v[0].at[pl.ds(0, n)],
                                    out_hbm.at[pl.ds(ci * N + base, n)])
            else:
                done = 0
                while done < _WB:
                    m = min(_GB, _WB - done)
                    off = pl.multiple_of(si * _WB + done, 8)
                    pltpu.sync_copy(acc_sh.at[pl.ds(off, m)],
                                    rows_v[0].at[pl.ds(0, m)])
                    pltpu.sync_copy(rows_v[0].at[pl.ds(0, m)],
                                    out_hbm.at[pl.ds(ci * N + off, m)])
                    done += m

    return pl.kernel(
        body,
        out_type=jax.ShapeDtypeStruct((_NC * N, H), jnp.float32),
        mesh=_mesh(),
        compiler_params=_PARAMS,
        scratch_types=[
            [[pltpu.VMEM((_SB,), jnp.int32) for _ in range(_NSUB)]
             for _ in range(nbuf)],
            [[pltpu.VMEM((_SB,), jnp.int32) for _ in range(_NSUB)]
             for _ in range(nbuf)],
            [pltpu.VMEM((_GB, H), jnp.float32) for _ in range(nbuf)],
            pltpu.VMEM_SHARED((N, H), jnp.float32),
            pltpu.SemaphoreType.DMA,
            [pltpu.SemaphoreType.DMA for _ in range(nbuf)],
            [pltpu.SemaphoreType.DMA for _ in range(nbuf)],
        ],
    )


def _make_deg(N, E):
    H = 16
    _NSUB = 10
    _GB = _SB * _NSUB
    assert E % _GB == 0
    ncht = E // _GB
    nch = (ncht + _NW - 1) // _NW

    nbuf = 2
    nu = (nch + nbuf - 1) // nbuf

    def body(row_hbm, out_hbm, sidx, buf_v, acc_sh, sem, sems):
        ci = lax.axis_index("c")
        si = lax.axis_index("s")
        wid = si * _NC + ci

        def zrow(i, _):
            buf_v[i, pl.ds(0, 16)] = jnp.zeros((16,), jnp.float32)
            return _
        lax.fori_loop(0, _WB, zrow, None)
        for base, n in _slabs(N):
            if base > 0:
                @pl.when(si == 0)
                def _():
                    pltpu.sync_copy(buf_v.at[pl.ds(0, n)],
                                    acc_sh.at[pl.ds(base, n)])
            else:
                off = pl.multiple_of(si * _WB, 8)
                pltpu.sync_copy(buf_v.at[pl.ds(0, _WB)],
                                acc_sh.at[pl.ds(off, _WB)])

        def onerow(i, _):
            buf_v[i, pl.ds(0, 16)] = jnp.ones((16,), jnp.float32)
            return _
        lax.fori_loop(0, _SB, onerow, None)
        plsc.subcore_barrier()

        def drain(b, pred):
            @pl.when(pred)
            def _():
                for j in range(_NSUB):
                    pltpu.make_async_copy(buf_v.at[pl.ds(0, _SB)],
                                          acc_sh.at[sidx[b][j]],
                                          sems[b]).wait()

        def step(u, _):
            for b in range(nbuf):
                c = (u * nbuf + b) * _NW + wid
                cprev = c - nbuf * _NW
                drain(b, (u > 0) & (cprev < ncht))

                @pl.when(c < ncht)
                def _():
                    eoff = pl.multiple_of(c * _GB, 8)
                    hs = [pltpu.async_copy(
                              row_hbm.at[pl.ds(eoff + j * _SB, _SB)],
                              sidx[b][j], sem)
                          for j in range(_NSUB)]
                    for j in range(_NSUB):
                        hs[j].wait()
                        pltpu.async_copy(buf_v.at[pl.ds(0, _SB)],
                                         acc_sh.at[sidx[b][j]], sems[b],
                                         add=True)
            return _
        lax.fori_loop(0, nu, step, None)
        for b in range(nbuf):
            clast = ((nu - 1) * nbuf + b) * _NW + wid
            drain(b, clast < ncht)
        plsc.subcore_barrier()

        for base, n in _slabs(N):
            if base > 0:
                @pl.when(si == 0)
                def _():
                    pltpu.sync_copy(acc_sh.at[pl.ds(base, n)],
                                    buf_v.at[pl.ds(0, n)])
                    pltpu.sync_copy(buf_v.at[pl.ds(0, n)],
                                    out_hbm.at[pl.ds(ci * N + base, n)])
            else:
                off = pl.multiple_of(si * _WB, 8)
                pltpu.sync_copy(acc_sh.at[pl.ds(off, _WB)],
                                buf_v.at[pl.ds(0, _WB)])
                pltpu.sync_copy(buf_v.at[pl.ds(0, _WB)],
                                out_hbm.at[pl.ds(ci * N + off, _WB)])

    return pl.kernel(
        body,
        out_type=jax.ShapeDtypeStruct((_NC * N, H), jnp.float32),
        mesh=_mesh(),
        compiler_params=_PARAMS,
        scratch_types=[
            [[pltpu.VMEM((_SB,), jnp.int32) for _ in range(_NSUB)]
             for _ in range(nbuf)],
            pltpu.VMEM((_WB, H), jnp.float32),
            pltpu.VMEM_SHARED((N, H), jnp.float32),
            pltpu.SemaphoreType.DMA,
            [pltpu.SemaphoreType.DMA for _ in range(nbuf)],
        ],
    )


_BLK = 1000


def _tc_dense1(N, D, H, K1, nb):
    def body(x_ref, w_ref, pt_ref, pb_ref, o_ref, y_ref, d_ref):
        deg = pt_ref[...][:, :1] + pb_ref[...][:, :1]
        d = jnp.where(deg > 0, lax.rsqrt(deg), 0.0)
        d_ref[...] = d
        for k in range(K1):
            o_ref[k] = jnp.dot(x_ref[...], w_ref[k],
                               preferred_element_type=jnp.float32)
        y_ref[...] = d * o_ref[K1 - 1]
    return pl.pallas_call(
        body,
        grid=(N // _BLK,),
        in_specs=[pl.BlockSpec((_BLK, D), lambda i: (i, 0)),
                  pl.BlockSpec((K1, D, H), lambda i: (0, 0, 0)),
                  pl.BlockSpec((_BLK, 16), lambda i: (i, 0)),
                  pl.BlockSpec((_BLK, 16), lambda i: (i + nb, 0))],
        out_specs=[pl.BlockSpec((K1, _BLK, H), lambda i: (0, i, 0)),
                   pl.BlockSpec((_BLK, H), lambda i: (i, 0)),
                   pl.BlockSpec((_BLK, 1), lambda i: (i, 0))],
        out_shape=[jax.ShapeDtypeStruct((K1, N, H), jnp.float32),
                   jax.ShapeDtypeStruct((N, H), jnp.float32),
                   jax.ShapeDtypeStruct((N, 1), jnp.float32)],
    )


def _tc_combine(N, H, nb, kcol):
    def body(z_ref, pt_ref, pb_ref, b2_ref, d_ref, obk_ref, oy_ref):
        u = d_ref[...] * (pt_ref[...] + pb_ref[...])
        bk = z_ref[0] - 2.0 * u - b2_ref[...]
        obk_ref[...] = bk
        oy_ref[...] = d_ref[...] * bk
    return pl.pallas_call(
        body,
        grid=(N // _BLK,),
        in_specs=[pl.BlockSpec((1, _BLK, H), lambda i: (kcol, i, 0)),
                  pl.BlockSpec((_BLK, H), lambda i: (i, 0)),
                  pl.BlockSpec((_BLK, H), lambda i: (i + nb, 0)),
                  pl.BlockSpec((_BLK, H), lambda i: (i, 0)),
                  pl.BlockSpec((_BLK, 1), lambda i: (i, 0))],
        out_specs=[pl.BlockSpec((_BLK, H), lambda i: (i, 0)),
                   pl.BlockSpec((_BLK, H), lambda i: (i, 0))],
        out_shape=[jax.ShapeDtypeStruct((N, H), jnp.float32),
                   jax.ShapeDtypeStruct((N, H), jnp.float32)],
    )


def _tc_final(N, H, C, nb):
    def body(z_ref, pt_ref, pb_ref, b2k_ref, d_ref, b1_ref, w2_ref, bb_ref,
             o_ref):
        hpre = (z_ref[0] - d_ref[...] * (pt_ref[...] + pb_ref[...])
                - b2k_ref[...] + b1_ref[...])
        h = jnp.maximum(hpre, 0.0)
        logits = jnp.dot(h, w2_ref[...],
                         preferred_element_type=jnp.float32) + bb_ref[...]
        m = jnp.max(logits, axis=1, keepdims=True)
        e = jnp.exp(logits - m)
        o_ref[...] = e / jnp.sum(e, axis=1, keepdims=True)
    return pl.pallas_call(
        body,
        grid=(N // _BLK,),
        in_specs=[pl.BlockSpec((1, _BLK, H), lambda i: (0, i, 0)),
                  pl.BlockSpec((_BLK, H), lambda i: (i, 0)),
                  pl.BlockSpec((_BLK, H), lambda i: (i + nb, 0)),
                  pl.BlockSpec((_BLK, H), lambda i: (i, 0)),
                  pl.BlockSpec((_BLK, 1), lambda i: (i, 0)),
                  pl.BlockSpec((1, H), lambda i: (0, 0)),
                  pl.BlockSpec((H, C), lambda i: (0, 0)),
                  pl.BlockSpec((1, C), lambda i: (0, 0))],
        out_specs=pl.BlockSpec((_BLK, C), lambda i: (i, 0)),
        out_shape=jax.ShapeDtypeStruct((N, C), jnp.float32),
    )


def kernel(x, edge_index, W1, b1, W2, b2):
    N, D = x.shape
    E = edge_index.shape[1]
    K1, _, H = W1.shape
    C = W2.shape[2]
    KH = K1 * H
    nb = N // _BLK
    row, col = edge_index[0], edge_index[1]

    partd = _make_deg(N, E)(row)

    z3d, y, dis = _tc_dense1(N, D, H, K1, nb)(x, W1, partd, partd)

    prop_call = _make_prop(N, E, H)

    bk2 = jnp.zeros((N, H), jnp.float32)
    prev = z3d[K1 - 1]
    for k in range(K1 - 2, 0, -1):
        p = prop_call(y, row, col)
        bk, y = _tc_combine(N, H, nb, k)(z3d, p, p, bk2, dis)
        bk2 = prev
        prev = bk
    p = prop_call(y, row, col)
    return _tc_final(N, H, C, nb)(
        z3d, p, p, bk2, dis, b1.reshape(1, H), W2[0], b2.reshape(1, C))
